# Initial kernel scaffold; baseline (speedup 1.0000x reference)
import jax
import jax.numpy as jnp
from jax.experimental import pallas as pl


def kernel(feats, pts, knn_idx, W_delta, b_delta, gamma_delta, beta_delta, W_feats, b_feats, gamma_feats, beta_feats, W_post, b_post, gamma_post, beta_post):
    raise NotImplementedError("write your pallas kernel here")


# speedup vs baseline: 29.6144x; 29.6144x over previous
import functools
import jax
import jax.numpy as jnp
from jax import lax
from jax.experimental import pallas as pl
from jax.experimental.pallas import tpu as pltpu
from jax.experimental.pallas import tpu_sc as plsc

EPS = 1e-5
NW = 32
LANES = 16


def _table_body(p_ref, f_ref, wd_ref, wf_ref, bf_ref, o_ref):
    u = jnp.dot(p_ref[...], wd_ref[...], preferred_element_type=jnp.float32)
    v = jnp.dot(f_ref[...], wf_ref[...], preferred_element_type=jnp.float32)
    v = v + bf_ref[...]
    o_ref[...] = jnp.concatenate([u, v], axis=1)


def _build_table(ptsT, featsT, wdT, wfT, bf):
    BN, Cin = featsT.shape
    g2 = 2 * wdT.shape[1]
    blk = 512
    grid = (BN // blk,)
    return pl.pallas_call(
        _table_body,
        grid=grid,
        in_specs=[
            pl.BlockSpec((blk, ptsT.shape[1]), lambda i: (i, 0)),
            pl.BlockSpec((blk, Cin), lambda i: (i, 0)),
            pl.BlockSpec(wdT.shape, lambda i: (0, 0)),
            pl.BlockSpec(wfT.shape, lambda i: (0, 0)),
            pl.BlockSpec(bf.shape, lambda i: (0, 0)),
        ],
        out_specs=pl.BlockSpec((blk, g2), lambda i: (i, 0)),
        out_shape=jax.ShapeDtypeStruct((BN, g2), jnp.float32),
    )(ptsT, featsT, wdT, wfT, bf)


def _gather_body(tbl_hbm, idx_hbm, bd_hbm, g1_hbm, g2_hbm, st_hbm,
                 idx_v, cent_v, rows_v, g1_v, g2_v, bd_v, stat_v, sem,
                 *, n_chunks, cpw):
    cid = lax.axis_index("c")
    sid = lax.axis_index("s")
    wid = sid * 2 + cid

    epw = n_chunks * 128
    pltpu.sync_copy(idx_hbm.at[pl.ds(wid * n_chunks, n_chunks)], idx_v)
    pltpu.sync_copy(tbl_hbm.at[pl.ds(wid * cpw, cpw)], cent_v)
    pltpu.sync_copy(bd_hbm, bd_v)
    bd0 = bd_v[pl.ds(0, 16)]
    bd1 = bd_v[pl.ds(16, 16)]
    zero = jnp.zeros((16,), jnp.float32)

    def chunk_body(j, accs):
        pltpu.async_copy(tbl_hbm.at[idx_v.at[j]], rows_v, sem).wait()

        def cen_body(c8, accs):
            nl = j * 8 + c8
            c0 = cent_v[nl, pl.ds(0, 16)]
            c1 = cent_v[nl, pl.ds(16, 16)]
            s1a, s1b, q1a, q1b, s2a, s2b, q2a, q2b = accs
            for e in range(16):
                r = c8 * 16 + e
                u0 = rows_v[r, pl.ds(0, 16)]
                u1 = rows_v[r, pl.ds(16, 16)]
                v0 = rows_v[r, pl.ds(32, 16)]
                v1 = rows_v[r, pl.ds(48, 16)]
                g1a = u0 - c0 + bd0
                g1b = u1 - c1 + bd1
                g1_v[r, pl.ds(0, 16)] = g1a
                g1_v[r, pl.ds(16, 16)] = g1b
                g2_v[r, pl.ds(0, 16)] = v0
                g2_v[r, pl.ds(16, 16)] = v1
                s1a = s1a + g1a
                s1b = s1b + g1b
                q1a = q1a + g1a * g1a
                q1b = q1b + g1b * g1b
                s2a = s2a + v0
                s2b = s2b + v1
                q2a = q2a + v0 * v0
                q2b = q2b + v1 * v1
            return (s1a, s1b, q1a, q1b, s2a, s2b, q2a, q2b)

        accs = lax.fori_loop(0, 8, cen_body, accs)
        base = wid * epw + j * 128
        pltpu.sync_copy(g1_v, g1_hbm.at[pl.ds(base, 128)])
        pltpu.sync_copy(g2_v, g2_hbm.at[pl.ds(base, 128)])
        return accs

    accs = lax.fori_loop(0, n_chunks, chunk_body, (zero,) * 8)
    for i in range(8):
        stat_v[i, pl.ds(0, 16)] = accs[i]
    pltpu.sync_copy(stat_v, st_hbm.at[wid])


def _gather_pass(tbl, idx2d, b_delta):
    BN = tbl.shape[0]
    n_rows = idx2d.shape[0]
    E = n_rows * 128
    n_chunks = n_rows // NW
    cpw = BN // NW
    mesh = plsc.VectorSubcoreMesh(core_axis_name="c", subcore_axis_name="s")
    body = functools.partial(_gather_body, n_chunks=n_chunks, cpw=cpw)
    f = pl.kernel(
        body,
        out_type=[
            jax.ShapeDtypeStruct((E, 32), jnp.float32),
            jax.ShapeDtypeStruct((E, 32), jnp.float32),
            jax.ShapeDtypeStruct((NW, 8, 16), jnp.float32),
        ],
        mesh=mesh,
        scratch_types=[
            pltpu.VMEM((n_chunks, 128), jnp.int32),
            pltpu.VMEM((cpw, 64), jnp.float32),
            pltpu.VMEM((128, 64), jnp.float32),
            pltpu.VMEM((128, 32), jnp.float32),
            pltpu.VMEM((128, 32), jnp.float32),
            pltpu.VMEM((32,), jnp.float32),
            pltpu.VMEM((8, 16), jnp.float32),
            pltpu.SemaphoreType.DMA,
        ],
        compiler_params=pltpu.CompilerParams(use_tc_tiling_on_sc=False),
    )
    return f(tbl, idx2d, b_delta)


def _mix_body(g1_ref, g2_ref, a1, s1, a2, s2, wbd, bp, y_ref, st_ref, acc):
    i = pl.program_id(0)

    @pl.when(i == 0)
    def _():
        acc[...] = jnp.zeros_like(acc)

    w1 = jnp.maximum(g1_ref[...] * a1[...] + s1[...], 0.0)
    w2 = jnp.maximum(g2_ref[...] * a2[...] + s2[...], 0.0)
    z = w1 * w2
    y = jnp.dot(z, wbd[...], preferred_element_type=jnp.float32) + bp[...]
    y_ref[...] = y
    acc[0:1, :] += jnp.sum(y, axis=0, keepdims=True)
    acc[1:2, :] += jnp.sum(y * y, axis=0, keepdims=True)

    @pl.when(i == pl.num_programs(0) - 1)
    def _():
        st_ref[...] = acc[...]


def _mix_pass(G1r, G2r, a1, s1, a2, s2, wbd, bp):
    R = G1r.shape[0]
    blk = 2048
    grid = (R // blk,)
    vec = pl.BlockSpec((1, 128), lambda i: (0, 0))
    return pl.pallas_call(
        _mix_body,
        grid=grid,
        in_specs=[
            pl.BlockSpec((blk, 128), lambda i: (i, 0)),
            pl.BlockSpec((blk, 128), lambda i: (i, 0)),
            vec, vec, vec, vec,
            pl.BlockSpec((128, 128), lambda i: (0, 0)),
            vec,
        ],
        out_specs=[
            pl.BlockSpec((blk, 128), lambda i: (i, 0)),
            pl.BlockSpec((8, 128), lambda i: (0, 0)),
        ],
        out_shape=[
            jax.ShapeDtypeStruct((R, 128), jnp.float32),
            jax.ShapeDtypeStruct((8, 128), jnp.float32),
        ],
        scratch_shapes=[pltpu.VMEM((8, 128), jnp.float32)],
    )(G1r, G2r, a1, s1, a2, s2, wbd, bp)


def _fold_body(y_ref, a3, s3, o_ref):
    y = jnp.maximum(y_ref[...] * a3[...] + s3[...], 0.0)
    rows = y.shape[0] // 4
    s = y.reshape(rows, 4, 128).sum(axis=1)
    o_ref[...] = s[:, 0:32] + s[:, 32:64] + s[:, 64:96] + s[:, 96:128]


def _fold_pass(y3r, a3, s3, BN):
    R = y3r.shape[0]
    blk = 4096
    grid = (R // blk,)
    vec = pl.BlockSpec((1, 128), lambda i: (0, 0))
    return pl.pallas_call(
        _fold_body,
        grid=grid,
        in_specs=[
            pl.BlockSpec((blk, 128), lambda i: (i, 0)),
            vec, vec,
        ],
        out_specs=pl.BlockSpec((blk // 4, 32), lambda i: (i, 0)),
        out_shape=jax.ShapeDtypeStruct((BN, 32), jnp.float32),
    )(y3r, a3, s3)


def _affine(sum_, sumsq, count, gamma, beta):
    mean = sum_ / count
    var = sumsq / count - mean * mean
    sc = gamma * lax.rsqrt(var + EPS)
    return sc, beta - sc * mean


def kernel(feats, pts, knn_idx,
           W_delta, b_delta, gamma_delta, beta_delta,
           W_feats, b_feats, gamma_feats, beta_feats,
           W_post, b_post, gamma_post, beta_post):
    B, Cin, N = feats.shape
    K = knn_idx.shape[-1]
    g = W_delta.shape[0]
    BN = B * N
    E = BN * K
    cnt = jnp.float32(E)

    ptsT = pts.transpose(0, 2, 1).reshape(BN, 3)
    ptsT = jnp.pad(ptsT, ((0, 0), (0, 5)))
    featsT = feats.transpose(0, 2, 1).reshape(BN, Cin)
    wdT = jnp.pad(W_delta, ((0, 0), (0, 5))).T
    wfT = W_feats.T
    bf = b_feats[None, :]

    idx_flat = (knn_idx.astype(jnp.int32)
                + (jnp.arange(B, dtype=jnp.int32) * N)[:, None, None])
    idx2d = idx_flat.reshape(E // 128, 128)

    tbl = _build_table(ptsT, featsT, wdT, wfT, bf)

    g1, g2, st = _gather_pass(tbl, idx2d, b_delta)

    parts = st.sum(axis=0)
    s1 = jnp.concatenate([parts[0], parts[1]])
    q1 = jnp.concatenate([parts[2], parts[3]])
    s2 = jnp.concatenate([parts[4], parts[5]])
    q2 = jnp.concatenate([parts[6], parts[7]])
    sc1, sh1 = _affine(s1, q1, cnt, gamma_delta, beta_delta)
    sc2, sh2 = _affine(s2, q2, cnt, gamma_feats, beta_feats)

    G1r = g1.reshape(E // 4, 128)
    G2r = g2.reshape(E // 4, 128)
    wbd = jnp.kron(jnp.eye(4, dtype=jnp.float32), W_post.T)
    t4 = lambda x: jnp.tile(x, 4)[None, :]
    y3r, st3 = _mix_pass(G1r, G2r, t4(sc1), t4(sh1), t4(sc2), t4(sh2),
                         wbd, t4(b_post))

    s3 = st3[0].reshape(4, g).sum(axis=0)
    q3 = st3[1].reshape(4, g).sum(axis=0)
    sc3, sh3 = _affine(s3, q3, cnt, gamma_post, beta_post)

    out = _fold_pass(y3r, t4(sc3), t4(sh3), BN)
    return out.reshape(B, N, g).transpose(0, 2, 1)
